# baseline (device time: 34558 ns/iter reference)
import jax
import jax.numpy as jnp
from jax import lax
from jax.experimental import pallas as pl
from jax.experimental.pallas import tpu as pltpu

N_DEV = 4


def _ring_allgather(x_shard, d_shard):
    m, n = x_shard.shape
    dm, dn = d_shard.shape

    def body(x_ref, d_ref, xg_ref, dg_ref, xsend, xrecv, dsend, drecv):
        my = lax.axis_index("i")
        left = lax.rem(my + (N_DEV - 1), N_DEV)
        right = lax.rem(my + 1, N_DEV)

        barrier = pltpu.get_barrier_semaphore()
        for nbr in (left, right):
            pl.semaphore_signal(
                barrier,
                inc=1,
                device_id=(nbr,),
                device_id_type=pl.DeviceIdType.MESH,
            )
        pl.semaphore_wait(barrier, 2)

        xg_ref[my] = x_ref[:, :]
        dg_ref[my] = d_ref[:, :]

        for h in range(N_DEV - 1):
            src_o = lax.rem(my + (N_DEV - h), N_DEV)
            x_rdma = pltpu.make_async_remote_copy(
                src_ref=xg_ref.at[src_o],
                dst_ref=xg_ref.at[src_o],
                send_sem=xsend.at[h],
                recv_sem=xrecv.at[h],
                device_id=(right,),
                device_id_type=pl.DeviceIdType.MESH,
            )
            d_rdma = pltpu.make_async_remote_copy(
                src_ref=dg_ref.at[src_o],
                dst_ref=dg_ref.at[src_o],
                send_sem=dsend.at[h],
                recv_sem=drecv.at[h],
                device_id=(right,),
                device_id_type=pl.DeviceIdType.MESH,
            )
            x_rdma.start()
            d_rdma.start()
            x_rdma.wait()
            d_rdma.wait()

    return pl.pallas_call(
        body,
        out_shape=(
            jax.ShapeDtypeStruct((N_DEV, m, n), x_shard.dtype),
            jax.ShapeDtypeStruct((N_DEV, dm, dn), d_shard.dtype),
        ),
        in_specs=[
            pl.BlockSpec(memory_space=pltpu.VMEM),
            pl.BlockSpec(memory_space=pltpu.VMEM),
        ],
        out_specs=(
            pl.BlockSpec(memory_space=pltpu.VMEM),
            pl.BlockSpec(memory_space=pltpu.VMEM),
        ),
        scratch_shapes=[
            pltpu.SemaphoreType.DMA((N_DEV - 1,)),
            pltpu.SemaphoreType.DMA((N_DEV - 1,)),
            pltpu.SemaphoreType.DMA((N_DEV - 1,)),
            pltpu.SemaphoreType.DMA((N_DEV - 1,)),
        ],
        compiler_params=pltpu.CompilerParams(collective_id=0),
    )(x_shard, d_shard)


def kernel(x, dest):
    m, n = x.shape
    d2 = dest.reshape(m // 128, 128)
    x_all, d_all = _ring_allgather(x, d2)
    x_all = x_all.reshape(N_DEV * m, n)
    dest_all = d_all.reshape(N_DEV * m)

    order = jnp.argsort(dest_all, stable=True)
    my = lax.axis_index("i")
    idx = lax.dynamic_slice(order, (my * m,), (m,))
    return jnp.take(x_all, idx, axis=0)


# device time: 27722 ns/iter; 1.2466x vs baseline; 1.2466x over previous
import jax
import jax.numpy as jnp
from jax import lax
from jax.experimental import pallas as pl
from jax.experimental.pallas import tpu as pltpu

N_DEV = 4


def _ring_allgather(x_shard, d_shard):
    m, n = x_shard.shape
    dm, dn = d_shard.shape

    def body(x_ref, d_ref, xg_ref, dg_ref, xsend, xrecv, dsend, drecv):
        import functools

        my = lax.axis_index("i")
        left = lax.rem(my + (N_DEV - 1), N_DEV)
        right = lax.rem(my + 1, N_DEV)
        diag = lax.rem(my + 2, N_DEV)
        peers = (right, left, diag)

        barrier = pltpu.get_barrier_semaphore()
        for nbr in peers:
            pl.semaphore_signal(
                barrier,
                inc=1,
                device_id=(nbr,),
                device_id_type=pl.DeviceIdType.MESH,
            )
        pl.semaphore_wait(barrier, N_DEV - 1)

        xg_ref[my] = x_ref[:, :]
        dg_ref[my] = d_ref[:, :]

        rdmas = []
        for k, tgt in enumerate(peers):
            x_rdma = pltpu.make_async_remote_copy(
                src_ref=xg_ref.at[my],
                dst_ref=xg_ref.at[my],
                send_sem=xsend.at[k],
                recv_sem=xrecv.at[k],
                device_id=(tgt,),
                device_id_type=pl.DeviceIdType.MESH,
            )
            d_rdma = pltpu.make_async_remote_copy(
                src_ref=dg_ref.at[my],
                dst_ref=dg_ref.at[my],
                send_sem=dsend.at[k],
                recv_sem=drecv.at[k],
                device_id=(tgt,),
                device_id_type=pl.DeviceIdType.MESH,
            )
            x_rdma.start()
            d_rdma.start()
            rdmas.append((x_rdma, d_rdma))
        for x_rdma, d_rdma in rdmas:
            x_rdma.wait()
            d_rdma.wait()

        @functools.partial(
            pl.run_scoped, second_barrier=pltpu.SemaphoreType.REGULAR
        )
        def _(second_barrier):
            for nbr in peers:
                pl.semaphore_signal(
                    second_barrier,
                    inc=1,
                    device_id=(nbr,),
                    device_id_type=pl.DeviceIdType.MESH,
                )
            pl.semaphore_wait(second_barrier, N_DEV - 1)

    return pl.pallas_call(
        body,
        out_shape=(
            jax.ShapeDtypeStruct((N_DEV, m, n), x_shard.dtype),
            jax.ShapeDtypeStruct((N_DEV, dm, dn), d_shard.dtype),
        ),
        in_specs=[
            pl.BlockSpec(memory_space=pltpu.VMEM),
            pl.BlockSpec(memory_space=pltpu.VMEM),
        ],
        out_specs=(
            pl.BlockSpec(memory_space=pltpu.VMEM),
            pl.BlockSpec(memory_space=pltpu.VMEM),
        ),
        scratch_shapes=[
            pltpu.SemaphoreType.DMA((N_DEV - 1,)),
            pltpu.SemaphoreType.DMA((N_DEV - 1,)),
            pltpu.SemaphoreType.DMA((N_DEV - 1,)),
            pltpu.SemaphoreType.DMA((N_DEV - 1,)),
        ],
        compiler_params=pltpu.CompilerParams(collective_id=0),
    )(x_shard, d_shard)


def kernel(x, dest):
    m, n = x.shape
    d2 = dest.reshape(m // 128, 128)
    x_all, d_all = _ring_allgather(x, d2)
    x_all = x_all.reshape(N_DEV * m, n)
    dest_all = d_all.reshape(N_DEV * m)

    order = jnp.argsort(dest_all, stable=True)
    my = lax.axis_index("i")
    idx = lax.dynamic_slice(order, (my * m,), (m,))
    return jnp.take(x_all, idx, axis=0)
